# baseline (speedup 1.0000x reference)
import functools

import jax
import jax.numpy as jnp
from jax import lax
from jax.experimental import pallas as pl
from jax.experimental.pallas import tpu as pltpu
from jax.experimental.pallas import tpu_sc as plsc

NC = 2
NS = 16
NW = NC * NS
CH = 128
SG = 5


def _mesh():
    return plsc.VectorSubcoreMesh(
        core_axis_name="c", subcore_axis_name="s",
        num_cores=NC, num_subcores=NS)


@functools.lru_cache(maxsize=None)
def _build_wide(batch, widths):
    bpw = batch // NW
    nch = bpw // CH
    nbuf = len(widths)

    sr = SG * 16
    dr = CH - sr

    out_type = tuple(
        jax.ShapeDtypeStruct((batch, w), jnp.float32) for w in widths)
    scratch = (
        [pltpu.VMEM((bpw,), jnp.int32)]
        + [pltpu.VMEM((sr, w), jnp.float32) for w in widths]
        + [pltpu.SemaphoreType.DMA, pltpu.SemaphoreType.DMA,
           pltpu.SemaphoreType.DMA]
    )

    @functools.partial(
        pl.kernel, out_type=out_type, scratch_types=scratch, mesh=_mesh())
    def k(idx_hbm, *refs):
        tabs = refs[:nbuf]
        outs = refs[nbuf:2 * nbuf]
        idx_v = refs[2 * nbuf]
        vbufs = refs[2 * nbuf + 1:2 * nbuf + 1 + nbuf]
        gsem = refs[-3]
        wsem = refs[-2]
        dsem = refs[-1]
        wid = lax.axis_index("s") * NC + lax.axis_index("c")
        base = wid * bpw
        pltpu.sync_copy(idx_hbm.at[pl.ds(base, bpw)], idx_v)

        def wb_descr(b, c):
            return pltpu.make_async_copy(
                vbufs[b], outs[b].at[pl.ds(base + c * CH, sr)], wsem)

        for c in range(nch):
            for b in range(nbuf):
                if c > 0:
                    wb_descr(b, c - 1).wait()

                def body(g, carry, b=b, c=c):
                    v = idx_v[pl.ds(c * CH + g * 16, 16)]
                    for kk in range(16):
                        r = v[kk]
                        pltpu.async_copy(
                            tabs[b].at[pl.ds(r, 1)],
                            vbufs[b].at[pl.ds(g * 16 + kk, 1)],
                            gsem)
                    return carry

                lax.fori_loop(0, SG, body, 0)

                def dbody(g, carry, b=b, c=c):
                    v = idx_v[pl.ds(c * CH + sr + g * 16, 16)]
                    for kk in range(16):
                        r = v[kk]
                        pltpu.async_copy(
                            tabs[b].at[pl.ds(r, 1)],
                            outs[b].at[pl.ds(base + c * CH + sr + g * 16 + kk,
                                             1)],
                            dsem)
                    return carry

                lax.fori_loop(0, (CH - sr) // 16, dbody, 0)

                pltpu.make_async_copy(
                    tabs[b].at[pl.ds(0, sr)], vbufs[b], gsem).wait()
                wb_descr(b, c).start()
        for b in range(nbuf):
            wb_descr(b, nch - 1).wait()
            pltpu.make_async_copy(
                tabs[b].at[pl.ds(0, nch * dr)],
                outs[b].at[pl.ds(base, nch * dr)], dsem).wait()

    return k


@functools.lru_cache(maxsize=None)
def _build_narrow(batch, nbuf):
    bpw = batch // NW
    nch = bpw // CH

    out_type = tuple(
        jax.ShapeDtypeStruct((batch,), jnp.float32) for _ in range(nbuf))
    scratch = (
        [pltpu.VMEM((nch, CH), jnp.int32)]
        + [pltpu.VMEM((bpw,), jnp.float32) for _ in range(nbuf)]
        + [pltpu.SemaphoreType.DMA]
    )

    @functools.partial(
        pl.kernel, out_type=out_type, scratch_types=scratch, mesh=_mesh(),
        compiler_params=pltpu.CompilerParams(use_tc_tiling_on_sc=False))
    def k(idx_hbm, *refs):
        tabs = refs[:nbuf]
        outs = refs[nbuf:2 * nbuf]
        idx_v = refs[2 * nbuf]
        rows = refs[2 * nbuf + 1:2 * nbuf + 1 + nbuf]
        sem = refs[-1]
        wid = lax.axis_index("s") * NC + lax.axis_index("c")
        pltpu.sync_copy(idx_hbm.at[pl.ds(wid * nch, nch)], idx_v)
        for j in range(nch):
            cps = [
                pltpu.async_copy(
                    tabs[b].at[idx_v.at[j]],
                    rows[b].at[pl.ds(j * CH, CH)],
                    sem)
                for b in range(nbuf)
            ]
            for c in cps:
                c.wait()
        for b in range(nbuf):
            pltpu.sync_copy(rows[b], outs[b].at[pl.ds(wid * bpw, bpw)])

    return k


def kernel(indices, obs_buffer, next_obs_buffer, acts_buffer, rewards_buffer,
           terminals_buffer, rew_vects_buffer, term_vects_buffer):
    batch = indices.shape[0]
    wide_tabs = (obs_buffer, acts_buffer, next_obs_buffer,
                 rew_vects_buffer, term_vects_buffer)
    widths = tuple(t.shape[1] for t in wide_tabs)
    kw = _build_wide(batch, widths)
    observations, actions, next_observations, reward_vectors, \
        terminal_vectors = kw(indices, *wide_tabs)

    kn = _build_narrow(batch, 2)
    idx2d = indices.reshape(batch // CH, CH)
    rewards, terminals = kn(
        idx2d,
        rewards_buffer.reshape(rewards_buffer.shape[0]),
        terminals_buffer.reshape(terminals_buffer.shape[0]))
    return (observations, actions, rewards.reshape(batch, 1),
            terminals.reshape(batch, 1), next_observations,
            reward_vectors, terminal_vectors)


# speedup vs baseline: 1.2730x; 1.2730x over previous
import functools

import jax
import jax.numpy as jnp
from jax import lax
from jax.experimental import pallas as pl
from jax.experimental.pallas import tpu as pltpu
from jax.experimental.pallas import tpu_sc as plsc

NC = 2
NS = 16
NW = NC * NS
CH = 128


def _mesh():
    return plsc.VectorSubcoreMesh(
        core_axis_name="c", subcore_axis_name="s",
        num_cores=NC, num_subcores=NS)


@functools.lru_cache(maxsize=None)
def _build_wide(batch, widths):
    bpw = batch // NW
    nch = bpw // CH
    nbuf = len(widths)

    out_type = tuple(
        jax.ShapeDtypeStruct((batch, w), jnp.float32) for w in widths)
    scratch = (
        [pltpu.VMEM((bpw,), jnp.int32)]
        + [pltpu.VMEM((CH, w), jnp.float32) for w in widths]
        + [pltpu.SemaphoreType.DMA, pltpu.SemaphoreType.DMA]
    )

    @functools.partial(
        pl.kernel, out_type=out_type, scratch_types=scratch, mesh=_mesh())
    def k(idx_hbm, *refs):
        tabs = refs[:nbuf]
        outs = refs[nbuf:2 * nbuf]
        idx_v = refs[2 * nbuf]
        vbufs = refs[2 * nbuf + 1:2 * nbuf + 1 + nbuf]
        gsem = refs[-2]
        wsem = refs[-1]
        wid = lax.axis_index("s") * NC + lax.axis_index("c")
        base = wid * bpw
        pltpu.sync_copy(idx_hbm.at[pl.ds(base, bpw)], idx_v)

        def wb_descr(b, c):
            return pltpu.make_async_copy(
                vbufs[b], outs[b].at[pl.ds(base + c * CH, CH)], wsem)

        def drain_and_wb(b, c):
            pltpu.make_async_copy(
                tabs[b].at[pl.ds(0, CH)], vbufs[b], gsem).wait()
            wb_descr(b, c).start()

        pend = None
        for c in range(nch):
            for b in range(nbuf):
                if c > 0:
                    wb_descr(b, c - 1).wait()

                def body(g, carry, b=b, c=c):
                    v = idx_v[pl.ds(c * CH + g * 16, 16)]
                    for kk in range(16):
                        r = v[kk]
                        pltpu.async_copy(
                            tabs[b].at[pl.ds(r, 1)],
                            vbufs[b].at[pl.ds(g * 16 + kk, 1)],
                            gsem)
                    return carry

                lax.fori_loop(0, CH // 16, body, 0)
                if pend is not None:
                    drain_and_wb(*pend)
                pend = (b, c)
        drain_and_wb(*pend)
        for b in range(nbuf):
            wb_descr(b, nch - 1).wait()

    return k


@functools.lru_cache(maxsize=None)
def _build_narrow(batch, nbuf):
    bpw = batch // NW
    nch = bpw // CH

    out_type = tuple(
        jax.ShapeDtypeStruct((batch,), jnp.float32) for _ in range(nbuf))
    scratch = (
        [pltpu.VMEM((nch, CH), jnp.int32)]
        + [pltpu.VMEM((bpw,), jnp.float32) for _ in range(nbuf)]
        + [pltpu.SemaphoreType.DMA]
    )

    @functools.partial(
        pl.kernel, out_type=out_type, scratch_types=scratch, mesh=_mesh(),
        compiler_params=pltpu.CompilerParams(use_tc_tiling_on_sc=False))
    def k(idx_hbm, *refs):
        tabs = refs[:nbuf]
        outs = refs[nbuf:2 * nbuf]
        idx_v = refs[2 * nbuf]
        rows = refs[2 * nbuf + 1:2 * nbuf + 1 + nbuf]
        sem = refs[-1]
        wid = lax.axis_index("s") * NC + lax.axis_index("c")
        pltpu.sync_copy(idx_hbm.at[pl.ds(wid * nch, nch)], idx_v)
        for j in range(nch):
            cps = [
                pltpu.async_copy(
                    tabs[b].at[idx_v.at[j]],
                    rows[b].at[pl.ds(j * CH, CH)],
                    sem)
                for b in range(nbuf)
            ]
            for c in cps:
                c.wait()
        for b in range(nbuf):
            pltpu.sync_copy(rows[b], outs[b].at[pl.ds(wid * bpw, bpw)])

    return k


def kernel(indices, obs_buffer, next_obs_buffer, acts_buffer, rewards_buffer,
           terminals_buffer, rew_vects_buffer, term_vects_buffer):
    batch = indices.shape[0]
    wide_tabs = (obs_buffer, acts_buffer, next_obs_buffer,
                 rew_vects_buffer, term_vects_buffer)
    widths = tuple(t.shape[1] for t in wide_tabs)
    kw = _build_wide(batch, widths)
    observations, actions, next_observations, reward_vectors, \
        terminal_vectors = kw(indices, *wide_tabs)

    kn = _build_narrow(batch, 2)
    idx2d = indices.reshape(batch // CH, CH)
    rewards, terminals = kn(
        idx2d,
        rewards_buffer.reshape(rewards_buffer.shape[0]),
        terminals_buffer.reshape(terminals_buffer.shape[0]))
    return (observations, actions, rewards.reshape(batch, 1),
            terminals.reshape(batch, 1), next_observations,
            reward_vectors, terminal_vectors)
